# initial kernel scaffold (unmeasured)
import jax
import jax.numpy as jnp
from jax import lax
from jax.experimental import pallas as pl
from jax.experimental.pallas import tpu as pltpu

N_DEV = 4


def kernel(x, w_mat):
    m_per, k = x.shape
    _, n_per = w_mat.shape
    m_glob = N_DEV * m_per

    def body(x_ref, w_ref, out_ref, comm_ref, send_sems, recv_sems,
             amax_src, amax_box, amax_send_sems, amax_recv_sems):
        my = lax.axis_index("i")
        left = lax.rem(my - 1 + N_DEV, N_DEV)
        right = lax.rem(my + 1, N_DEV)

        barrier_sem = pltpu.get_barrier_semaphore()
        for nbr in [left, right]:
            pl.semaphore_signal(
                barrier_sem, inc=1,
                device_id=(nbr,), device_id_type=pl.DeviceIdType.MESH,
            )
        pl.semaphore_wait(barrier_sem, 2)

        out_ref[pl.ds(my * m_per, m_per), :] = jnp.dot(
            x_ref[:, :], w_ref[:, :], preferred_element_type=jnp.float32
        )
        comm_ref[0] = x_ref[:, :]

        for h in range(N_DEV - 1):
            send_slot = h % 2
            recv_slot = (h + 1) % 2
            rdma = pltpu.make_async_remote_copy(
                src_ref=comm_ref.at[send_slot],
                dst_ref=comm_ref.at[recv_slot],
                send_sem=send_sems.at[send_slot],
                recv_sem=recv_sems.at[recv_slot],
                device_id=(right,),
                device_id_type=pl.DeviceIdType.MESH,
            )
            rdma.start()
            rdma.wait()
            origin = lax.rem(my - h - 1 + N_DEV, N_DEV)
            out_ref[pl.ds(origin * m_per, m_per), :] = jnp.dot(
                comm_ref[recv_slot], w_ref[:, :],
                preferred_element_type=jnp.float32,
            )

        local_amax = jnp.maximum(jnp.max(out_ref[:, :]), 0.0)
        amax_src[:, :] = jnp.full((8, 128), local_amax, jnp.float32)

        send_descs = []
        for d in range(1, N_DEV):
            peer = lax.rem(my + d, N_DEV)
            s = pltpu.make_async_remote_copy(
                src_ref=amax_src,
                dst_ref=amax_box.at[my],
                send_sem=amax_send_sems.at[d],
                recv_sem=amax_recv_sems.at[my],
                device_id=(peer,),
                device_id_type=pl.DeviceIdType.MESH,
            )
            s.start()
            send_descs.append(s)

        g_amax = local_amax
        for d in range(1, N_DEV):
            peer = lax.rem(my + d, N_DEV)
            r = pltpu.make_async_remote_copy(
                src_ref=amax_src,
                dst_ref=amax_box.at[peer],
                send_sem=amax_send_sems.at[d],
                recv_sem=amax_recv_sems.at[peer],
                device_id=(peer,),
                device_id_type=pl.DeviceIdType.MESH,
            )
            r.wait_recv()
            g_amax = jnp.maximum(g_amax, amax_box[peer, 0, 0])
        for s in send_descs:
            s.wait_send()

        scale = g_amax / 127.0
        y = jnp.maximum(out_ref[:, :], 0.0)
        q = jnp.clip(jnp.round(y / scale), -127.0, 127.0)
        out_ref[:, :] = q * scale

    return pl.pallas_call(
        body,
        out_shape=jax.ShapeDtypeStruct((m_glob, n_per), jnp.float32),
        in_specs=[
            pl.BlockSpec(memory_space=pltpu.VMEM),
            pl.BlockSpec(memory_space=pltpu.VMEM),
        ],
        out_specs=pl.BlockSpec(memory_space=pltpu.VMEM),
        scratch_shapes=[
            pltpu.VMEM((2, m_per, k), jnp.float32),
            pltpu.SemaphoreType.DMA((2,)),
            pltpu.SemaphoreType.DMA((2,)),
            pltpu.VMEM((8, 128), jnp.float32),
            pltpu.VMEM((N_DEV, 8, 128), jnp.float32),
            pltpu.SemaphoreType.DMA((N_DEV,)),
            pltpu.SemaphoreType.DMA((N_DEV,)),
        ],
        compiler_params=pltpu.CompilerParams(collective_id=0),
    )(x, w_mat)


# baseline (device time: 591656 ns/iter reference)
import jax
import jax.numpy as jnp
from jax import lax
from jax.experimental import pallas as pl
from jax.experimental.pallas import tpu as pltpu

N_DEV = 4


def kernel(x, w_mat):
    m_per, k = x.shape
    _, n_per = w_mat.shape
    m_glob = N_DEV * m_per

    def body(x_ref, w_ref, out_ref, comm_ref, send_sems, recv_sems,
             amax_src, amax_box, amax_send_sems, amax_recv_sems, local_sem):
        my = lax.axis_index("i")
        left = lax.rem(my - 1 + N_DEV, N_DEV)
        right = lax.rem(my + 1, N_DEV)

        barrier_sem = pltpu.get_barrier_semaphore()
        for nbr in [left, right]:
            pl.semaphore_signal(
                barrier_sem, inc=1,
                device_id=(nbr,), device_id_type=pl.DeviceIdType.MESH,
            )
        pl.semaphore_wait(barrier_sem, 2)

        cp = pltpu.make_async_copy(x_ref, comm_ref.at[0], local_sem)
        cp.start()
        cp.wait()

        out_ref[pl.ds(my * m_per, m_per), :] = jnp.dot(
            comm_ref[0], w_ref[:, :], preferred_element_type=jnp.float32
        )

        for h in range(N_DEV - 1):
            send_slot = h % 2
            recv_slot = (h + 1) % 2
            rdma = pltpu.make_async_remote_copy(
                src_ref=comm_ref.at[send_slot],
                dst_ref=comm_ref.at[recv_slot],
                send_sem=send_sems.at[send_slot],
                recv_sem=recv_sems.at[recv_slot],
                device_id=(right,),
                device_id_type=pl.DeviceIdType.MESH,
            )
            rdma.start()
            rdma.wait()
            origin = lax.rem(my - h - 1 + N_DEV, N_DEV)
            out_ref[pl.ds(origin * m_per, m_per), :] = jnp.dot(
                comm_ref[recv_slot], w_ref[:, :],
                preferred_element_type=jnp.float32,
            )

        local_amax = jnp.maximum(jnp.max(out_ref[:, :]), 0.0)
        amax_src[:, :] = jnp.full((8, 128), local_amax, jnp.float32)

        send_descs = []
        for d in range(1, N_DEV):
            peer = lax.rem(my + d, N_DEV)
            s = pltpu.make_async_remote_copy(
                src_ref=amax_src,
                dst_ref=amax_box.at[my],
                send_sem=amax_send_sems.at[d],
                recv_sem=amax_recv_sems.at[my],
                device_id=(peer,),
                device_id_type=pl.DeviceIdType.MESH,
            )
            s.start()
            send_descs.append(s)

        g_amax = local_amax
        for d in range(1, N_DEV):
            peer = lax.rem(my + d, N_DEV)
            r = pltpu.make_async_remote_copy(
                src_ref=amax_src,
                dst_ref=amax_box.at[peer],
                send_sem=amax_send_sems.at[d],
                recv_sem=amax_recv_sems.at[peer],
                device_id=(peer,),
                device_id_type=pl.DeviceIdType.MESH,
            )
            r.wait_recv()
            g_amax = jnp.maximum(g_amax, amax_box[peer, 0, 0])
        for s in send_descs:
            s.wait_send()

        scale = g_amax / 127.0
        y = jnp.maximum(out_ref[:, :], 0.0)
        q = jnp.clip(jnp.round(y / scale), -127.0, 127.0)
        out_ref[:, :] = q * scale

    return pl.pallas_call(
        body,
        out_shape=jax.ShapeDtypeStruct((m_glob, n_per), jnp.float32),
        in_specs=[
            pl.BlockSpec(memory_space=pl.ANY),
            pl.BlockSpec(memory_space=pltpu.VMEM),
        ],
        out_specs=pl.BlockSpec(memory_space=pltpu.VMEM),
        scratch_shapes=[
            pltpu.VMEM((2, m_per, k), jnp.float32),
            pltpu.SemaphoreType.DMA((2,)),
            pltpu.SemaphoreType.DMA((2,)),
            pltpu.VMEM((8, 128), jnp.float32),
            pltpu.VMEM((N_DEV, 8, 128), jnp.float32),
            pltpu.SemaphoreType.DMA((N_DEV,)),
            pltpu.SemaphoreType.DMA((N_DEV,)),
            pltpu.SemaphoreType.DMA,
        ],
        compiler_params=pltpu.CompilerParams(
            collective_id=0,
            vmem_limit_bytes=100 * 1024 * 1024,
        ),
    )(x, w_mat)


# device time: 307507 ns/iter; 1.9240x vs baseline; 1.9240x over previous
import jax
import jax.numpy as jnp
from jax import lax
from jax.experimental import pallas as pl
from jax.experimental.pallas import tpu as pltpu

N_DEV = 4


def kernel(x, w_mat):
    m_per, k = x.shape
    _, n_per = w_mat.shape
    m_glob = N_DEV * m_per
    kh = k // 2

    def body(x_ref, w_ref, out_ref, comm_r, comm_l, send_r, recv_r,
             send_l, recv_l, amax_src, amax_box, amax_send_sems,
             amax_recv_sems, local_sems):
        my = lax.axis_index("i")
        left = lax.rem(my - 1 + N_DEV, N_DEV)
        right = lax.rem(my + 1, N_DEV)

        cpA = pltpu.make_async_copy(
            x_ref.at[:, pl.ds(0, kh)], comm_r.at[0], local_sems.at[0])
        cpB = pltpu.make_async_copy(
            x_ref.at[:, pl.ds(kh, kh)], comm_l.at[0], local_sems.at[1])
        cpA.start()
        cpB.start()
        cpA.wait()
        cpB.wait()

        barrier_sem = pltpu.get_barrier_semaphore()
        for nbr in [left, right]:
            pl.semaphore_signal(
                barrier_sem, inc=1,
                device_id=(nbr,), device_id_type=pl.DeviceIdType.MESH,
            )
        pl.semaphore_wait(barrier_sem, 2)

        def dotA(chunk):
            return jnp.dot(chunk, w_ref[pl.ds(0, kh), :],
                           preferred_element_type=jnp.float32)

        def dotB(chunk):
            return jnp.dot(chunk, w_ref[pl.ds(kh, kh), :],
                           preferred_element_type=jnp.float32)

        def rows(o):
            return pl.ds(lax.rem(o + 2 * N_DEV, N_DEV) * m_per, m_per)

        def ring_start(h):
            sslot, rslot = h % 2, (h + 1) % 2
            r = pltpu.make_async_remote_copy(
                src_ref=comm_r.at[sslot], dst_ref=comm_r.at[rslot],
                send_sem=send_r.at[sslot], recv_sem=recv_r.at[rslot],
                device_id=(right,), device_id_type=pl.DeviceIdType.MESH)
            l = pltpu.make_async_remote_copy(
                src_ref=comm_l.at[sslot], dst_ref=comm_l.at[rslot],
                send_sem=send_l.at[sslot], recv_sem=recv_l.at[rslot],
                device_id=(left,), device_id_type=pl.DeviceIdType.MESH)
            r.start()
            l.start()
            return r, l

        r0, l0 = ring_start(0)
        out_ref[rows(my), :] = dotA(comm_r[0]) + dotB(comm_l[0])
        r0.wait()
        l0.wait()

        r1, l1 = ring_start(1)
        out_ref[rows(my - 1), :] = dotA(comm_r[1])
        out_ref[rows(my + 1), :] = dotB(comm_l[1])
        r1.wait()
        l1.wait()

        r2, l2 = ring_start(2)
        out_ref[rows(my + 2), :] = dotA(comm_r[0]) + dotB(comm_l[0])
        r2.wait()
        l2.wait()

        out_ref[rows(my + 1), :] = out_ref[rows(my + 1), :] + dotA(comm_r[1])
        out_ref[rows(my - 1), :] = out_ref[rows(my - 1), :] + dotB(comm_l[1])

        local_amax = jnp.maximum(jnp.max(out_ref[:, :]), 0.0)
        amax_src[:, :] = jnp.full((8, 128), local_amax, jnp.float32)

        send_descs = []
        for d in range(1, N_DEV):
            peer = lax.rem(my + d, N_DEV)
            s = pltpu.make_async_remote_copy(
                src_ref=amax_src,
                dst_ref=amax_box.at[my],
                send_sem=amax_send_sems.at[d],
                recv_sem=amax_recv_sems.at[my],
                device_id=(peer,),
                device_id_type=pl.DeviceIdType.MESH,
            )
            s.start()
            send_descs.append(s)

        g_amax = local_amax
        for d in range(1, N_DEV):
            peer = lax.rem(my + d, N_DEV)
            r = pltpu.make_async_remote_copy(
                src_ref=amax_src,
                dst_ref=amax_box.at[peer],
                send_sem=amax_send_sems.at[d],
                recv_sem=amax_recv_sems.at[peer],
                device_id=(peer,),
                device_id_type=pl.DeviceIdType.MESH,
            )
            r.wait_recv()
            g_amax = jnp.maximum(g_amax, amax_box[peer, 0, 0])
        for s in send_descs:
            s.wait_send()

        scale = g_amax / 127.0
        y = jnp.maximum(out_ref[:, :], 0.0)
        q = jnp.clip(jnp.round(y / scale), -127.0, 127.0)
        out_ref[:, :] = q * scale

    return pl.pallas_call(
        body,
        out_shape=jax.ShapeDtypeStruct((m_glob, n_per), jnp.float32),
        in_specs=[
            pl.BlockSpec(memory_space=pl.ANY),
            pl.BlockSpec(memory_space=pltpu.VMEM),
        ],
        out_specs=pl.BlockSpec(memory_space=pltpu.VMEM),
        scratch_shapes=[
            pltpu.VMEM((2, m_per, kh), jnp.float32),
            pltpu.VMEM((2, m_per, kh), jnp.float32),
            pltpu.SemaphoreType.DMA((2,)),
            pltpu.SemaphoreType.DMA((2,)),
            pltpu.SemaphoreType.DMA((2,)),
            pltpu.SemaphoreType.DMA((2,)),
            pltpu.VMEM((8, 128), jnp.float32),
            pltpu.VMEM((N_DEV, 8, 128), jnp.float32),
            pltpu.SemaphoreType.DMA((N_DEV,)),
            pltpu.SemaphoreType.DMA((N_DEV,)),
            pltpu.SemaphoreType.DMA((2,)),
        ],
        compiler_params=pltpu.CompilerParams(
            collective_id=0,
            vmem_limit_bytes=100 * 1024 * 1024,
        ),
    )(x, w_mat)


# device time: 203955 ns/iter; 2.9009x vs baseline; 1.5077x over previous
import jax
import jax.numpy as jnp
from jax import lax
from jax.experimental import pallas as pl
from jax.experimental.pallas import tpu as pltpu

N_DEV = 4


def kernel(x, w_mat):
    m_per, k = x.shape
    _, n_per = w_mat.shape
    m_glob = N_DEV * m_per
    nh = n_per // 2

    def body(x_hbm, w_ref, out_ref, x_vmem, comm_r, comm_l, piece_buf,
             send_r, recv_r, send_l, recv_l, piece_send_sems,
             piece_recv_sems, amax_src, amax_box, amax_send_sems,
             amax_recv_sems, local_sems):
        my = lax.axis_index("i")
        left = lax.rem(my - 1 + N_DEV, N_DEV)
        right = lax.rem(my + 1, N_DEV)

        MESH = pl.DeviceIdType.MESH

        cpx = pltpu.make_async_copy(x_hbm, x_vmem, local_sems.at[0])
        cpA = pltpu.make_async_copy(
            w_ref.at[:, pl.ds(0, nh)], comm_r.at[0], local_sems.at[1])
        cpB = pltpu.make_async_copy(
            w_ref.at[:, pl.ds(nh, nh)], comm_l.at[0], local_sems.at[2])
        cpx.start()
        cpA.start()
        cpB.start()
        cpA.wait()
        cpB.wait()

        barrier_sem = pltpu.get_barrier_semaphore()
        for nbr in [left, right]:
            pl.semaphore_signal(barrier_sem, inc=1, device_id=(nbr,),
                                device_id_type=MESH)
        pl.semaphore_wait(barrier_sem, 2)

        def rows(o):
            return pl.ds(lax.rem(o + 2 * N_DEV, N_DEV) * m_per, m_per)

        def ring_start(h):
            sslot, rslot = h % 2, (h + 1) % 2
            r = pltpu.make_async_remote_copy(
                src_ref=comm_r.at[sslot], dst_ref=comm_r.at[rslot],
                send_sem=send_r.at[sslot], recv_sem=recv_r.at[rslot],
                device_id=(right,), device_id_type=MESH)
            l = pltpu.make_async_remote_copy(
                src_ref=comm_l.at[sslot], dst_ref=comm_l.at[rslot],
                send_sem=send_l.at[sslot], recv_sem=recv_l.at[rslot],
                device_id=(left,), device_id_type=MESH)
            r.start()
            l.start()
            return r, l

        piece_sends = []

        def piece(slot, w_half_chunk, dest, hh):
            piece_buf[slot] = jnp.dot(x_vmem[:, :], w_half_chunk,
                                      preferred_element_type=jnp.float32)
            s = pltpu.make_async_remote_copy(
                src_ref=piece_buf.at[slot],
                dst_ref=out_ref.at[rows(my), pl.ds(hh * nh, nh)],
                send_sem=piece_send_sems.at[slot],
                recv_sem=piece_recv_sems.at[my, hh],
                device_id=(lax.rem(dest + 2 * N_DEV, N_DEV),),
                device_id_type=MESH)
            s.start()
            piece_sends.append(s)
            return jnp.max(piece_buf[slot])

        r0, l0 = ring_start(0)
        cpx.wait()
        out_ref[rows(my), :] = jnp.dot(x_vmem[:, :], w_ref[:, :],
                                       preferred_element_type=jnp.float32)
        g_amax = jnp.maximum(jnp.max(out_ref[rows(my), :]), 0.0)
        r0.wait()
        l0.wait()

        r1, l1 = ring_start(1)
        g_amax = jnp.maximum(g_amax, piece(0, comm_r[1], my - 1, 0))
        g_amax = jnp.maximum(g_amax, piece(1, comm_l[1], my + 1, 1))
        r1.wait()
        l1.wait()

        r2, l2 = ring_start(2)
        g_amax = jnp.maximum(g_amax, piece(2, comm_r[0], my - 2, 0))
        g_amax = jnp.maximum(g_amax, piece(3, comm_l[0], my + 2, 1))
        r2.wait()
        l2.wait()

        g_amax = jnp.maximum(g_amax, piece(4, comm_r[1], my + 1, 0))
        g_amax = jnp.maximum(g_amax, piece(5, comm_l[1], my - 1, 1))

        amax_src[:, :] = jnp.full((8, 128), g_amax, jnp.float32)
        amax_sends = []
        for d in range(1, N_DEV):
            peer = lax.rem(my + d, N_DEV)
            s = pltpu.make_async_remote_copy(
                src_ref=amax_src,
                dst_ref=amax_box.at[my],
                send_sem=amax_send_sems.at[d],
                recv_sem=amax_recv_sems.at[my],
                device_id=(peer,), device_id_type=MESH)
            s.start()
            amax_sends.append(s)
        for d in range(1, N_DEV):
            peer = lax.rem(my + d, N_DEV)
            r = pltpu.make_async_remote_copy(
                src_ref=amax_src,
                dst_ref=amax_box.at[peer],
                send_sem=amax_send_sems.at[d],
                recv_sem=amax_recv_sems.at[peer],
                device_id=(peer,), device_id_type=MESH)
            r.wait_recv()
            g_amax = jnp.maximum(g_amax, amax_box[peer, 0, 0])
        for s in amax_sends:
            s.wait_send()

        for d in range(1, N_DEV):
            sender = lax.rem(my + d, N_DEV)
            for hh in range(2):
                rcv = pltpu.make_async_remote_copy(
                    src_ref=piece_buf.at[0],
                    dst_ref=out_ref.at[rows(sender), pl.ds(hh * nh, nh)],
                    send_sem=piece_send_sems.at[0],
                    recv_sem=piece_recv_sems.at[sender, hh],
                    device_id=(sender,), device_id_type=MESH)
                rcv.wait_recv()
        for s in piece_sends:
            s.wait_send()

        scale = g_amax / 127.0
        y = jnp.maximum(out_ref[:, :], 0.0)
        q = jnp.clip(jnp.round(y / scale), -127.0, 127.0)
        out_ref[:, :] = q * scale

    return pl.pallas_call(
        body,
        out_shape=jax.ShapeDtypeStruct((m_glob, n_per), jnp.float32),
        in_specs=[
            pl.BlockSpec(memory_space=pl.ANY),
            pl.BlockSpec(memory_space=pltpu.VMEM),
        ],
        out_specs=pl.BlockSpec(memory_space=pltpu.VMEM),
        scratch_shapes=[
            pltpu.VMEM((m_per, k), jnp.float32),
            pltpu.VMEM((2, k, nh), jnp.float32),
            pltpu.VMEM((2, k, nh), jnp.float32),
            pltpu.VMEM((6, m_per, nh), jnp.float32),
            pltpu.SemaphoreType.DMA((2,)),
            pltpu.SemaphoreType.DMA((2,)),
            pltpu.SemaphoreType.DMA((2,)),
            pltpu.SemaphoreType.DMA((2,)),
            pltpu.SemaphoreType.DMA((6,)),
            pltpu.SemaphoreType.DMA((N_DEV, 2)),
            pltpu.VMEM((8, 128), jnp.float32),
            pltpu.VMEM((N_DEV, 8, 128), jnp.float32),
            pltpu.SemaphoreType.DMA((N_DEV,)),
            pltpu.SemaphoreType.DMA((N_DEV,)),
            pltpu.SemaphoreType.DMA((3,)),
        ],
        compiler_params=pltpu.CompilerParams(
            collective_id=0,
            vmem_limit_bytes=100 * 1024 * 1024,
        ),
    )(x, w_mat)


# device time: 202248 ns/iter; 2.9254x vs baseline; 1.0084x over previous
import jax
import jax.numpy as jnp
from jax import lax
from jax.experimental import pallas as pl
from jax.experimental.pallas import tpu as pltpu

N_DEV = 4


def kernel(x, w_mat):
    m_per, k = x.shape
    _, n_per = w_mat.shape
    m_glob = N_DEV * m_per
    nh = n_per // 2
    nq = n_per // 4

    def body(x_hbm, w_ref, out_ref, x_vmem, comm_r, comm_l, piece_buf,
             send_r, recv_r, send_l, recv_l, piece_send_sems,
             piece_recv_sems, amax_src, amax_box, amax_send_sems,
             amax_recv_sems, local_sem):
        my = lax.axis_index("i")
        left = lax.rem(my - 1 + N_DEV, N_DEV)
        right = lax.rem(my + 1, N_DEV)

        MESH = pl.DeviceIdType.MESH

        cpx = pltpu.make_async_copy(x_hbm, x_vmem, local_sem)
        cpx.start()

        barrier_sem = pltpu.get_barrier_semaphore()
        for nbr in [left, right]:
            pl.semaphore_signal(barrier_sem, inc=1, device_id=(nbr,),
                                device_id_type=MESH)
        pl.semaphore_wait(barrier_sem, 2)

        def rows(o):
            return pl.ds(lax.rem(o + 2 * N_DEV, N_DEV) * m_per, m_per)

        def ring_start(h):
            sslot, rslot = h % 2, (h + 1) % 2
            src_r = comm_r.at[sslot] if h else w_ref.at[:, pl.ds(0, nh)]
            src_l = comm_l.at[sslot] if h else w_ref.at[:, pl.ds(nh, nh)]
            r = pltpu.make_async_remote_copy(
                src_ref=src_r, dst_ref=comm_r.at[rslot],
                send_sem=send_r.at[sslot], recv_sem=recv_r.at[rslot],
                device_id=(right,), device_id_type=MESH)
            l = pltpu.make_async_remote_copy(
                src_ref=src_l, dst_ref=comm_l.at[rslot],
                send_sem=send_l.at[sslot], recv_sem=recv_l.at[rslot],
                device_id=(left,), device_id_type=MESH)
            r.start()
            l.start()
            return r, l

        piece_sends = []

        def send_piece(src, dst_col, width, dest, sem_slot, recv_q):
            s = pltpu.make_async_remote_copy(
                src_ref=src,
                dst_ref=out_ref.at[rows(my), pl.ds(dst_col, width)],
                send_sem=piece_send_sems.at[sem_slot],
                recv_sem=piece_recv_sems.at[my, recv_q],
                device_id=(lax.rem(dest + 2 * N_DEV, N_DEV),),
                device_id_type=MESH)
            s.start()
            piece_sends.append(s)

        def piece(slot, w_half_chunk, dest, hh):
            piece_buf[slot] = jnp.dot(x_vmem[:, :], w_half_chunk,
                                      preferred_element_type=jnp.float32)
            send_piece(piece_buf.at[slot], hh * nh, nh, dest, slot, 2 * hh)
            return jnp.max(piece_buf[slot])

        r0, l0 = ring_start(0)
        cpx.wait()
        out_ref[rows(my), :] = jnp.dot(x_vmem[:, :], w_ref[:, :],
                                       preferred_element_type=jnp.float32)
        g_amax = jnp.maximum(jnp.max(out_ref[rows(my), :]), 0.0)
        r0.wait()
        l0.wait()

        r1, l1 = ring_start(1)
        g_amax = jnp.maximum(g_amax, piece(0, comm_r[1], my - 1, 0))
        g_amax = jnp.maximum(g_amax, piece(1, comm_l[1], my + 1, 1))
        r1.wait()
        l1.wait()

        r2, l2 = ring_start(2)
        g_amax = jnp.maximum(g_amax, piece(2, comm_r[0], my - 2, 0))
        g_amax = jnp.maximum(g_amax, piece(3, comm_l[0], my + 2, 1))
        r2.wait()
        l2.wait()

        for sub in range(2):
            piece_buf[4, :, pl.ds(sub * nq, nq)] = jnp.dot(
                x_vmem[:, :], comm_r[1, :, pl.ds(sub * nq, nq)],
                preferred_element_type=jnp.float32)
            send_piece(piece_buf.at[4, :, pl.ds(sub * nq, nq)],
                       sub * nq, nq, my + 1, 4 + sub, sub)
            piece_buf[5, :, pl.ds(sub * nq, nq)] = jnp.dot(
                x_vmem[:, :], comm_l[1, :, pl.ds(sub * nq, nq)],
                preferred_element_type=jnp.float32)
            send_piece(piece_buf.at[5, :, pl.ds(sub * nq, nq)],
                       nh + sub * nq, nq, my - 1, 6 + sub, 2 + sub)
        g_amax = jnp.maximum(g_amax, jnp.max(piece_buf[4]))
        g_amax = jnp.maximum(g_amax, jnp.max(piece_buf[5]))

        amax_src[:, :] = jnp.full((8, 128), g_amax, jnp.float32)
        amax_sends = []
        for d in range(1, N_DEV):
            peer = lax.rem(my + d, N_DEV)
            s = pltpu.make_async_remote_copy(
                src_ref=amax_src,
                dst_ref=amax_box.at[my],
                send_sem=amax_send_sems.at[d],
                recv_sem=amax_recv_sems.at[my],
                device_id=(peer,), device_id_type=MESH)
            s.start()
            amax_sends.append(s)
        for d in range(1, N_DEV):
            peer = lax.rem(my + d, N_DEV)
            r = pltpu.make_async_remote_copy(
                src_ref=amax_src,
                dst_ref=amax_box.at[peer],
                send_sem=amax_send_sems.at[d],
                recv_sem=amax_recv_sems.at[peer],
                device_id=(peer,), device_id_type=MESH)
            r.wait_recv()
            g_amax = jnp.maximum(g_amax, amax_box[peer, 0, 0])

        scale = g_amax / 127.0

        def epilogue(band):
            y = jnp.maximum(out_ref[band, :], 0.0)
            q = jnp.clip(jnp.round(y / scale), -127.0, 127.0)
            out_ref[band, :] = q * scale

        def wait_piece(sender, col, width, q):
            rcv = pltpu.make_async_remote_copy(
                src_ref=piece_buf.at[0, :, pl.ds(0, width)],
                dst_ref=out_ref.at[rows(sender), pl.ds(col, width)],
                send_sem=piece_send_sems.at[0],
                recv_sem=piece_recv_sems.at[sender, q],
                device_id=(sender,), device_id_type=MESH)
            rcv.wait_recv()

        epilogue(rows(my))

        s2 = lax.rem(my + 2, N_DEV)
        wait_piece(s2, 0, nh, 0)
        wait_piece(s2, nh, nh, 2)
        epilogue(rows(my + 2))

        s1 = lax.rem(my + 1, N_DEV)
        wait_piece(s1, 0, nh, 0)
        wait_piece(s1, nh, nq, 2)
        wait_piece(s1, nh + nq, nq, 3)
        epilogue(rows(my + 1))

        s3 = lax.rem(my - 1 + N_DEV, N_DEV)
        wait_piece(s3, nh, nh, 2)
        wait_piece(s3, 0, nq, 0)
        wait_piece(s3, nq, nq, 1)
        epilogue(rows(my - 1))

        for s in amax_sends:
            s.wait_send()
        for s in piece_sends:
            s.wait_send()

    return pl.pallas_call(
        body,
        out_shape=jax.ShapeDtypeStruct((m_glob, n_per), jnp.float32),
        in_specs=[
            pl.BlockSpec(memory_space=pl.ANY),
            pl.BlockSpec(memory_space=pltpu.VMEM),
        ],
        out_specs=pl.BlockSpec(memory_space=pltpu.VMEM),
        scratch_shapes=[
            pltpu.VMEM((m_per, k), jnp.float32),
            pltpu.VMEM((2, k, nh), jnp.float32),
            pltpu.VMEM((2, k, nh), jnp.float32),
            pltpu.VMEM((6, m_per, nh), jnp.float32),
            pltpu.SemaphoreType.DMA((2,)),
            pltpu.SemaphoreType.DMA((2,)),
            pltpu.SemaphoreType.DMA((2,)),
            pltpu.SemaphoreType.DMA((2,)),
            pltpu.SemaphoreType.DMA((8,)),
            pltpu.SemaphoreType.DMA((N_DEV, 4)),
            pltpu.VMEM((8, 128), jnp.float32),
            pltpu.VMEM((N_DEV, 8, 128), jnp.float32),
            pltpu.SemaphoreType.DMA((N_DEV,)),
            pltpu.SemaphoreType.DMA((N_DEV,)),
            pltpu.SemaphoreType.DMA,
        ],
        compiler_params=pltpu.CompilerParams(
            collective_id=0,
            vmem_limit_bytes=100 * 1024 * 1024,
        ),
    )(x, w_mat)


# device time: 186293 ns/iter; 3.1759x vs baseline; 1.0856x over previous
import jax
import jax.numpy as jnp
from jax import lax
from jax.experimental import pallas as pl
from jax.experimental.pallas import tpu as pltpu

N_DEV = 4


def kernel(x, w_mat):
    m_per, k = x.shape
    _, n_per = w_mat.shape
    m_glob = N_DEV * m_per
    nh = n_per // 2
    nq = n_per // 4

    def body(x_hbm, w_ref, out_ref, x_vmem, comm_r, comm_l, piece_buf,
             recv_y, send_r, recv_r, send_l, recv_l, piece_send_sems,
             piece_recv_sems, amax_src, amax_box, amax_send_sems,
             amax_recv_sems, local_sem):
        my = lax.axis_index("i")
        left = lax.rem(my - 1 + N_DEV, N_DEV)
        right = lax.rem(my + 1, N_DEV)

        MESH = pl.DeviceIdType.MESH

        cpx = pltpu.make_async_copy(x_hbm, x_vmem, local_sem)
        cpx.start()

        barrier_sem = pltpu.get_barrier_semaphore()
        for nbr in [left, right]:
            pl.semaphore_signal(barrier_sem, inc=1, device_id=(nbr,),
                                device_id_type=MESH)
        pl.semaphore_wait(barrier_sem, 2)

        def rows(o):
            return pl.ds(lax.rem(o + 2 * N_DEV, N_DEV) * m_per, m_per)

        def ring_start(h):
            sslot, rslot = h % 2, (h + 1) % 2
            src_r = comm_r.at[sslot] if h else w_ref.at[:, pl.ds(0, nh)]
            src_l = comm_l.at[sslot] if h else w_ref.at[:, pl.ds(nh, nh)]
            r = pltpu.make_async_remote_copy(
                src_ref=src_r, dst_ref=comm_r.at[rslot],
                send_sem=send_r.at[sslot], recv_sem=recv_r.at[rslot],
                device_id=(right,), device_id_type=MESH)
            l = pltpu.make_async_remote_copy(
                src_ref=src_l, dst_ref=comm_l.at[rslot],
                send_sem=send_l.at[sslot], recv_sem=recv_l.at[rslot],
                device_id=(left,), device_id_type=MESH)
            r.start()
            l.start()
            return r, l

        piece_sends = []

        def send_piece(src, dst_col, width, dest, sem_slot, recv_q):
            s = pltpu.make_async_remote_copy(
                src_ref=src,
                dst_ref=recv_y.at[my, :, pl.ds(dst_col, width)],
                send_sem=piece_send_sems.at[sem_slot],
                recv_sem=piece_recv_sems.at[my, recv_q],
                device_id=(lax.rem(dest + 2 * N_DEV, N_DEV),),
                device_id_type=MESH)
            s.start()
            piece_sends.append(s)

        def piece(slot, w_half_chunk, dest, hh):
            d = jnp.dot(x_vmem[:, :], w_half_chunk,
                        preferred_element_type=jnp.float32)
            piece_buf[slot] = d.astype(jnp.bfloat16)
            send_piece(piece_buf.at[slot], hh * nh, nh, dest, slot, 2 * hh)
            return jnp.max(d)

        r0, l0 = ring_start(0)
        cpx.wait()
        out_ref[rows(my), :] = jnp.dot(x_vmem[:, :], w_ref[:, :],
                                       preferred_element_type=jnp.float32)
        g_amax = jnp.maximum(jnp.max(out_ref[rows(my), :]), 0.0)
        r0.wait()
        l0.wait()

        r1, l1 = ring_start(1)
        g_amax = jnp.maximum(g_amax, piece(0, comm_r[1], my - 1, 0))
        g_amax = jnp.maximum(g_amax, piece(1, comm_l[1], my + 1, 1))
        r1.wait()
        l1.wait()

        r2, l2 = ring_start(2)
        g_amax = jnp.maximum(g_amax, piece(2, comm_r[0], my - 2, 0))
        g_amax = jnp.maximum(g_amax, piece(3, comm_l[0], my + 2, 1))
        r2.wait()
        l2.wait()

        for sub in range(2):
            dA = jnp.dot(x_vmem[:, :], comm_r[1, :, pl.ds(sub * nq, nq)],
                         preferred_element_type=jnp.float32)
            piece_buf[4, :, pl.ds(sub * nq, nq)] = dA.astype(jnp.bfloat16)
            send_piece(piece_buf.at[4, :, pl.ds(sub * nq, nq)],
                       sub * nq, nq, my + 1, 4 + sub, sub)
            dB = jnp.dot(x_vmem[:, :], comm_l[1, :, pl.ds(sub * nq, nq)],
                         preferred_element_type=jnp.float32)
            piece_buf[5, :, pl.ds(sub * nq, nq)] = dB.astype(jnp.bfloat16)
            send_piece(piece_buf.at[5, :, pl.ds(sub * nq, nq)],
                       nh + sub * nq, nq, my - 1, 6 + sub, 2 + sub)
            g_amax = jnp.maximum(g_amax, jnp.max(dA))
            g_amax = jnp.maximum(g_amax, jnp.max(dB))

        amax_src[:, :] = jnp.full((8, 128), g_amax, jnp.float32)
        amax_sends = []
        for d in range(1, N_DEV):
            peer = lax.rem(my + d, N_DEV)
            s = pltpu.make_async_remote_copy(
                src_ref=amax_src,
                dst_ref=amax_box.at[my],
                send_sem=amax_send_sems.at[d],
                recv_sem=amax_recv_sems.at[my],
                device_id=(peer,), device_id_type=MESH)
            s.start()
            amax_sends.append(s)
        for d in range(1, N_DEV):
            peer = lax.rem(my + d, N_DEV)
            r = pltpu.make_async_remote_copy(
                src_ref=amax_src,
                dst_ref=amax_box.at[peer],
                send_sem=amax_send_sems.at[d],
                recv_sem=amax_recv_sems.at[peer],
                device_id=(peer,), device_id_type=MESH)
            r.wait_recv()
            g_amax = jnp.maximum(g_amax, amax_box[peer, 0, 0])

        scale = g_amax / 127.0

        def epilogue_own(band):
            y = jnp.maximum(out_ref[band, :], 0.0)
            q = jnp.clip(jnp.round(y / scale), -127.0, 127.0)
            out_ref[band, :] = q * scale

        def epilogue_recv(sender):
            y = jnp.maximum(recv_y[sender].astype(jnp.float32), 0.0)
            q = jnp.clip(jnp.round(y / scale), -127.0, 127.0)
            out_ref[rows(sender), :] = q * scale

        def wait_piece(sender, col, width, q):
            rcv = pltpu.make_async_remote_copy(
                src_ref=piece_buf.at[0, :, pl.ds(0, width)],
                dst_ref=recv_y.at[sender, :, pl.ds(col, width)],
                send_sem=piece_send_sems.at[0],
                recv_sem=piece_recv_sems.at[sender, q],
                device_id=(sender,), device_id_type=MESH)
            rcv.wait_recv()

        epilogue_own(rows(my))

        s2 = lax.rem(my + 2, N_DEV)
        wait_piece(s2, 0, nh, 0)
        wait_piece(s2, nh, nh, 2)
        epilogue_recv(s2)

        s1 = lax.rem(my + 1, N_DEV)
        wait_piece(s1, 0, nh, 0)
        wait_piece(s1, nh, nq, 2)
        wait_piece(s1, nh + nq, nq, 3)
        epilogue_recv(s1)

        s3 = lax.rem(my - 1 + N_DEV, N_DEV)
        wait_piece(s3, nh, nh, 2)
        wait_piece(s3, 0, nq, 0)
        wait_piece(s3, nq, nq, 1)
        epilogue_recv(s3)

        for s in amax_sends:
            s.wait_send()
        for s in piece_sends:
            s.wait_send()

    return pl.pallas_call(
        body,
        out_shape=jax.ShapeDtypeStruct((m_glob, n_per), jnp.float32),
        in_specs=[
            pl.BlockSpec(memory_space=pl.ANY),
            pl.BlockSpec(memory_space=pltpu.VMEM),
        ],
        out_specs=pl.BlockSpec(memory_space=pltpu.VMEM),
        scratch_shapes=[
            pltpu.VMEM((m_per, k), jnp.float32),
            pltpu.VMEM((2, k, nh), jnp.float32),
            pltpu.VMEM((2, k, nh), jnp.float32),
            pltpu.VMEM((6, m_per, nh), jnp.bfloat16),
            pltpu.VMEM((N_DEV, m_per, n_per), jnp.bfloat16),
            pltpu.SemaphoreType.DMA((2,)),
            pltpu.SemaphoreType.DMA((2,)),
            pltpu.SemaphoreType.DMA((2,)),
            pltpu.SemaphoreType.DMA((2,)),
            pltpu.SemaphoreType.DMA((8,)),
            pltpu.SemaphoreType.DMA((N_DEV, 4)),
            pltpu.VMEM((8, 128), jnp.float32),
            pltpu.VMEM((N_DEV, 8, 128), jnp.float32),
            pltpu.SemaphoreType.DMA((N_DEV,)),
            pltpu.SemaphoreType.DMA((N_DEV,)),
            pltpu.SemaphoreType.DMA,
        ],
        compiler_params=pltpu.CompilerParams(
            collective_id=0,
            vmem_limit_bytes=100 * 1024 * 1024,
        ),
    )(x, w_mat)


# device time: 180778 ns/iter; 3.2728x vs baseline; 1.0305x over previous
import jax
import jax.numpy as jnp
from jax import lax
from jax.experimental import pallas as pl
from jax.experimental.pallas import tpu as pltpu

N_DEV = 4


def kernel(x, w_mat):
    m_per, k = x.shape
    _, n_per = w_mat.shape
    m_glob = N_DEV * m_per
    nh = n_per // 2
    nq = n_per // 4

    def body(x_hbm, w_ref, out_ref, x_vmem, comm_r, comm_l, piece_buf,
             recv_y, send_r, recv_r, send_l, recv_l, piece_send_sems,
             piece_recv_sems, amax_src, amax_box, amax_send_sems,
             amax_recv_sems, local_sem):
        my = lax.axis_index("i")
        left = lax.rem(my - 1 + N_DEV, N_DEV)
        right = lax.rem(my + 1, N_DEV)

        MESH = pl.DeviceIdType.MESH

        cpx = pltpu.make_async_copy(x_hbm, x_vmem, local_sem)
        cpx.start()

        barrier_sem = pltpu.get_barrier_semaphore()
        for nbr in [left, right]:
            pl.semaphore_signal(barrier_sem, inc=1, device_id=(nbr,),
                                device_id_type=MESH)
        pl.semaphore_wait(barrier_sem, 2)

        def rows(o):
            return pl.ds(lax.rem(o + 2 * N_DEV, N_DEV) * m_per, m_per)

        def ring_start(h):
            sslot, rslot = h % 2, (h + 1) % 2
            src_r = comm_r.at[sslot] if h else w_ref.at[:, pl.ds(0, nh)]
            src_l = comm_l.at[sslot] if h else w_ref.at[:, pl.ds(nh, nh)]
            r = pltpu.make_async_remote_copy(
                src_ref=src_r, dst_ref=comm_r.at[rslot],
                send_sem=send_r.at[sslot], recv_sem=recv_r.at[rslot],
                device_id=(right,), device_id_type=MESH)
            l = pltpu.make_async_remote_copy(
                src_ref=src_l, dst_ref=comm_l.at[rslot],
                send_sem=send_l.at[sslot], recv_sem=recv_l.at[rslot],
                device_id=(left,), device_id_type=MESH)
            r.start()
            l.start()
            return r, l

        piece_sends = []

        def send_piece(src, dst_col, width, dest, sem_slot, recv_q):
            s = pltpu.make_async_remote_copy(
                src_ref=src,
                dst_ref=recv_y.at[my, :, pl.ds(dst_col, width)],
                send_sem=piece_send_sems.at[sem_slot],
                recv_sem=piece_recv_sems.at[my, recv_q],
                device_id=(lax.rem(dest + 2 * N_DEV, N_DEV),),
                device_id_type=MESH)
            s.start()
            piece_sends.append(s)

        def piece(slot, w_half_chunk, dest, hh):
            d = jnp.dot(x_vmem[:, :], w_half_chunk,
                        preferred_element_type=jnp.float32)
            piece_buf[slot] = d.astype(jnp.bfloat16)
            send_piece(piece_buf.at[slot], hh * nh, nh, dest, slot, 2 * hh)
            return jnp.max(d)

        r0, l0 = ring_start(0)
        cpx.wait()
        out_ref[rows(my), :] = jnp.dot(x_vmem[:, :], w_ref[:, :],
                                       preferred_element_type=jnp.float32)
        g_amax = jnp.maximum(jnp.max(out_ref[rows(my), :]), 0.0)
        r0.wait()
        l0.wait()

        r1, l1 = ring_start(1)
        g_amax = jnp.maximum(g_amax, piece(0, comm_r[1], my - 1, 0))
        g_amax = jnp.maximum(g_amax, piece(1, comm_l[1], my + 1, 1))
        r1.wait()
        l1.wait()

        def sub_desc(buf, sub, ssem, rsem, dev):
            return pltpu.make_async_remote_copy(
                src_ref=buf.at[0, :, pl.ds(sub * nq, nq)],
                dst_ref=buf.at[1, :, pl.ds(sub * nq, nq)],
                send_sem=ssem, recv_sem=rsem,
                device_id=(dev,), device_id_type=MESH)

        r2a = sub_desc(comm_r, 0, send_r.at[0], recv_r.at[1], right)
        r2b = sub_desc(comm_r, 1, send_r.at[2], recv_r.at[2], right)
        l2a = sub_desc(comm_l, 0, send_l.at[0], recv_l.at[1], left)
        l2b = sub_desc(comm_l, 1, send_l.at[2], recv_l.at[2], left)
        r2a.start()
        l2a.start()
        r2b.start()
        l2b.start()

        g_amax = jnp.maximum(g_amax, piece(2, comm_r[0], my - 2, 0))
        g_amax = jnp.maximum(g_amax, piece(3, comm_l[0], my + 2, 1))

        def tail_piece(buf_slot, comm, sub, dst_col, dest, sem_slot, recv_q):
            d = jnp.dot(x_vmem[:, :], comm[1, :, pl.ds(sub * nq, nq)],
                        preferred_element_type=jnp.float32)
            piece_buf[buf_slot, :, pl.ds(sub * nq, nq)] = (
                d.astype(jnp.bfloat16))
            send_piece(piece_buf.at[buf_slot, :, pl.ds(sub * nq, nq)],
                       dst_col, nq, dest, sem_slot, recv_q)
            return jnp.max(d)

        r2a.wait()
        g_amax = jnp.maximum(g_amax, tail_piece(4, comm_r, 0, 0, my + 1, 4, 0))
        l2a.wait()
        g_amax = jnp.maximum(
            g_amax, tail_piece(5, comm_l, 0, nh, my - 1, 6, 2))
        r2b.wait()
        g_amax = jnp.maximum(
            g_amax, tail_piece(4, comm_r, 1, nq, my + 1, 5, 1))
        l2b.wait()
        g_amax = jnp.maximum(
            g_amax, tail_piece(5, comm_l, 1, nh + nq, my - 1, 7, 3))

        amax_src[:, :] = jnp.full((8, 128), g_amax, jnp.float32)
        amax_sends = []
        for d in range(1, N_DEV):
            peer = lax.rem(my + d, N_DEV)
            s = pltpu.make_async_remote_copy(
                src_ref=amax_src,
                dst_ref=amax_box.at[my],
                send_sem=amax_send_sems.at[d],
                recv_sem=amax_recv_sems.at[my],
                device_id=(peer,), device_id_type=MESH)
            s.start()
            amax_sends.append(s)
        for d in range(1, N_DEV):
            peer = lax.rem(my + d, N_DEV)
            r = pltpu.make_async_remote_copy(
                src_ref=amax_src,
                dst_ref=amax_box.at[peer],
                send_sem=amax_send_sems.at[d],
                recv_sem=amax_recv_sems.at[peer],
                device_id=(peer,), device_id_type=MESH)
            r.wait_recv()
            g_amax = jnp.maximum(g_amax, amax_box[peer, 0, 0])

        scale = g_amax / 127.0

        def epilogue_own(band):
            y = jnp.maximum(out_ref[band, :], 0.0)
            q = jnp.clip(jnp.round(y / scale), -127.0, 127.0)
            out_ref[band, :] = q * scale

        def epilogue_recv(sender):
            y = jnp.maximum(recv_y[sender].astype(jnp.float32), 0.0)
            q = jnp.clip(jnp.round(y / scale), -127.0, 127.0)
            out_ref[rows(sender), :] = q * scale

        def wait_piece(sender, col, width, q):
            rcv = pltpu.make_async_remote_copy(
                src_ref=piece_buf.at[0, :, pl.ds(0, width)],
                dst_ref=recv_y.at[sender, :, pl.ds(col, width)],
                send_sem=piece_send_sems.at[0],
                recv_sem=piece_recv_sems.at[sender, q],
                device_id=(sender,), device_id_type=MESH)
            rcv.wait_recv()

        epilogue_own(rows(my))

        s2 = lax.rem(my + 2, N_DEV)
        wait_piece(s2, 0, nh, 0)
        wait_piece(s2, nh, nh, 2)
        epilogue_recv(s2)

        s1 = lax.rem(my + 1, N_DEV)
        wait_piece(s1, 0, nh, 0)
        wait_piece(s1, nh, nq, 2)
        wait_piece(s1, nh + nq, nq, 3)
        epilogue_recv(s1)

        s3 = lax.rem(my - 1 + N_DEV, N_DEV)
        wait_piece(s3, nh, nh, 2)
        wait_piece(s3, 0, nq, 0)
        wait_piece(s3, nq, nq, 1)
        epilogue_recv(s3)

        for s in amax_sends:
            s.wait_send()
        for s in piece_sends:
            s.wait_send()

    return pl.pallas_call(
        body,
        out_shape=jax.ShapeDtypeStruct((m_glob, n_per), jnp.float32),
        in_specs=[
            pl.BlockSpec(memory_space=pl.ANY),
            pl.BlockSpec(memory_space=pltpu.VMEM),
        ],
        out_specs=pl.BlockSpec(memory_space=pltpu.VMEM),
        scratch_shapes=[
            pltpu.VMEM((m_per, k), jnp.float32),
            pltpu.VMEM((2, k, nh), jnp.float32),
            pltpu.VMEM((2, k, nh), jnp.float32),
            pltpu.VMEM((6, m_per, nh), jnp.bfloat16),
            pltpu.VMEM((N_DEV, m_per, n_per), jnp.bfloat16),
            pltpu.SemaphoreType.DMA((3,)),
            pltpu.SemaphoreType.DMA((3,)),
            pltpu.SemaphoreType.DMA((3,)),
            pltpu.SemaphoreType.DMA((3,)),
            pltpu.SemaphoreType.DMA((8,)),
            pltpu.SemaphoreType.DMA((N_DEV, 4)),
            pltpu.VMEM((8, 128), jnp.float32),
            pltpu.VMEM((N_DEV, 8, 128), jnp.float32),
            pltpu.SemaphoreType.DMA((N_DEV,)),
            pltpu.SemaphoreType.DMA((N_DEV,)),
            pltpu.SemaphoreType.DMA,
        ],
        compiler_params=pltpu.CompilerParams(
            collective_id=0,
            vmem_limit_bytes=100 * 1024 * 1024,
        ),
    )(x, w_mat)
